# baseline (speedup 1.0000x reference)
import jax
import jax.numpy as jnp
import numpy as np
from jax.experimental import pallas as pl
from jax.experimental.pallas import tpu as pltpu

_TB = 2048


def _molu(x):
    return 0.5 * x * (1.0 + jnp.tanh(x))


def _fwd_kernel(x_ref, w1_ref, b1_ref, w2_ref, b2_ref,
                fw1_ref, fb1_ref, fw2_ref, fb2_ref,
                o_ref, y1_ref, f_ref):
    for py in range(12):
        win = x_ref[56 * py:56 * py + 168, :]
        m = jnp.dot(w1_ref[...], win,
                    preferred_element_type=jnp.float32)
        p = jnp.maximum(m[0:240, :], m[240:480, :])
        p = jnp.maximum(p[0:120, :], p[120:240, :])
        y1_ref[120 * py:120 * (py + 1), :] = _molu(p + b1_ref[...])

    for py2 in range(4):
        win = y1_ref[240 * py2:240 * py2 + 720, :]
        m = jnp.dot(w2_ref[...], win,
                    preferred_element_type=jnp.float32)
        p = jnp.maximum(m[0:160, :], m[160:320, :])
        p = jnp.maximum(p[0:80, :], p[80:160, :])
        f_ref[80 * py2:80 * (py2 + 1), :] = p

    feats = _molu(f_ref[...] + b2_ref[...])

    h = _molu(jnp.dot(fw1_ref[...], feats,
                      preferred_element_type=jnp.float32) + fb1_ref[...])
    logits = jnp.dot(fw2_ref[...], h,
                     preferred_element_type=jnp.float32) + fb2_ref[...]
    mx = jnp.max(logits, axis=0, keepdims=True)
    sh = logits - mx
    lse = jnp.log(jnp.sum(jnp.exp(sh), axis=0, keepdims=True))
    o_ref[...] = (sh - lse).astype(o_ref.dtype)


def _run(x_flat, w1b, b1c, w2b, b2c, fw1p, fb1c, fw2m, fb2c):
    n_pad = x_flat.shape[-1]
    tb = _TB
    grid = (n_pad // tb,)
    return pl.pallas_call(
        _fwd_kernel,
        out_shape=jax.ShapeDtypeStruct((10, n_pad), jnp.float32),
        grid=grid,
        in_specs=[
            pl.BlockSpec((784, tb), lambda i: (0, i)),
            pl.BlockSpec((480, 168), lambda i: (0, 0)),
            pl.BlockSpec((120, 1), lambda i: (0, 0)),
            pl.BlockSpec((320, 720), lambda i: (0, 0)),
            pl.BlockSpec((320, 1), lambda i: (0, 0)),
            pl.BlockSpec((50, 320), lambda i: (0, 0)),
            pl.BlockSpec((50, 1), lambda i: (0, 0)),
            pl.BlockSpec((10, 50), lambda i: (0, 0)),
            pl.BlockSpec((10, 1), lambda i: (0, 0)),
        ],
        out_specs=pl.BlockSpec((10, tb), lambda i: (0, i)),
        scratch_shapes=[
            pltpu.VMEM((1440, tb), jnp.float32),
            pltpu.VMEM((320, tb), jnp.float32),
        ],
        compiler_params=pltpu.CompilerParams(
            dimension_semantics=("parallel",),
            vmem_limit_bytes=40 * 1024 * 1024,
        ),
    )(x_flat, w1b, b1c, w2b, b2c, fw1p, fb1c, fw2m, fb2c)


def _row_onehot(n_r):
    a = np.zeros((2, n_r, 5), np.float32)
    for d in range(2):
        for h in range(5):
            a[d, d + h, h] = 1.0
    return a


def _col_onehot(n_p, n_w):
    b = np.zeros((2 * n_p, n_w, 5), np.float32)
    for d in range(2):
        for p in range(n_p):
            for k in range(5):
                b[d * n_p + p, 2 * p + d + k, k] = 1.0
    return b


_A1 = _row_onehot(6)
_B1 = _col_onehot(12, 28)
_A2 = _row_onehot(6)
_B2 = _col_onehot(4, 12)


def _pack_conv1(w1):
    return jnp.einsum("drh,qwk,chk->dqcrw", _A1, _B1,
                      w1[:, 0]).reshape(480, 168)


def _pack_conv2(w2):
    return jnp.einsum("drh,qwk,oihk->dqorwi", _A2, _B2,
                      w2).reshape(320, 720)


@jax.jit
def _forward(x_nchw, w1, b1, w2, b2, fw1, fb1, fw2, fb2):
    n = x_nchw.shape[0]
    tb = _TB
    n_pad = ((n + tb - 1) // tb) * tb

    x = x_nchw.reshape(n, 784).astype(jnp.float32)
    if n_pad != n:
        x = jnp.pad(x, ((0, n_pad - n), (0, 0)))
    x_flat = x.T

    w1b = _pack_conv1(w1.astype(jnp.float32))
    b1c = jnp.tile(b1.astype(jnp.float32), 12).reshape(120, 1)
    w2b = _pack_conv2(w2.astype(jnp.float32))
    b2c = jnp.tile(b2.astype(jnp.float32), 16).reshape(320, 1)
    fw1p = (fw1.astype(jnp.float32).reshape(50, 20, 4, 4)
            .transpose(0, 2, 3, 1).reshape(50, 320))
    fb1c = fb1.reshape(50, 1).astype(jnp.float32)
    fw2m = fw2.astype(jnp.float32)
    fb2c = fb2.reshape(10, 1).astype(jnp.float32)

    out = _run(x_flat, w1b, b1c, w2b, b2c, fw1p, fb1c, fw2m, fb2c)
    return out[:, :n].T


def kernel(x_nchw, w1, b1, w2, b2, fw1, fb1, fw2, fb2):
    return _forward(x_nchw, w1, b1, w2, b2, fw1, fb1, fw2, fb2)


# speedup vs baseline: 1.0177x; 1.0177x over previous
import jax
import jax.numpy as jnp
import numpy as np
from jax.experimental import pallas as pl
from jax.experimental.pallas import tpu as pltpu

_TB = 1024


def _molu(x):
    return 0.5 * x * (1.0 + jnp.tanh(x))


def _fwd_kernel(x_ref, w1_ref, b1_ref, w2_ref, b2_ref,
                fw1_ref, fb1_ref, fw2_ref, fb2_ref,
                o_ref, y1_ref, f_ref):
    for py in range(12):
        win = x_ref[56 * py:56 * py + 168, :].astype(jnp.bfloat16)
        m = jnp.dot(w1_ref[...], win,
                    preferred_element_type=jnp.float32)
        p = jnp.maximum(m[0:240, :], m[240:480, :])
        p = jnp.maximum(p[0:120, :], p[120:240, :])
        y1_ref[120 * py:120 * (py + 1), :] = _molu(p + b1_ref[...]).astype(jnp.bfloat16)

    for py2 in range(4):
        win = y1_ref[240 * py2:240 * py2 + 720, :]
        m = jnp.dot(w2_ref[...], win,
                    preferred_element_type=jnp.float32)
        p = jnp.maximum(m[0:160, :], m[160:320, :])
        p = jnp.maximum(p[0:80, :], p[80:160, :])
        f_ref[80 * py2:80 * (py2 + 1), :] = p

    feats = _molu(f_ref[...] + b2_ref[...])

    h = _molu(jnp.dot(fw1_ref[...], feats.astype(jnp.bfloat16),
                      preferred_element_type=jnp.float32) + fb1_ref[...])
    logits = jnp.dot(fw2_ref[...], h.astype(jnp.bfloat16),
                     preferred_element_type=jnp.float32) + fb2_ref[...]
    mx = jnp.max(logits, axis=0, keepdims=True)
    sh = logits - mx
    lse = jnp.log(jnp.sum(jnp.exp(sh), axis=0, keepdims=True))
    o_ref[...] = (sh - lse).astype(o_ref.dtype)


def _run(x_flat, w1b, b1c, w2b, b2c, fw1p, fb1c, fw2m, fb2c):
    n_pad = x_flat.shape[-1]
    tb = _TB
    grid = (n_pad // tb,)
    return pl.pallas_call(
        _fwd_kernel,
        out_shape=jax.ShapeDtypeStruct((10, n_pad), jnp.float32),
        grid=grid,
        in_specs=[
            pl.BlockSpec((784, tb), lambda i: (0, i)),
            pl.BlockSpec((480, 168), lambda i: (0, 0)),
            pl.BlockSpec((120, 1), lambda i: (0, 0)),
            pl.BlockSpec((320, 720), lambda i: (0, 0)),
            pl.BlockSpec((320, 1), lambda i: (0, 0)),
            pl.BlockSpec((50, 320), lambda i: (0, 0)),
            pl.BlockSpec((50, 1), lambda i: (0, 0)),
            pl.BlockSpec((10, 50), lambda i: (0, 0)),
            pl.BlockSpec((10, 1), lambda i: (0, 0)),
        ],
        out_specs=pl.BlockSpec((10, tb), lambda i: (0, i)),
        scratch_shapes=[
            pltpu.VMEM((1440, tb), jnp.bfloat16),
            pltpu.VMEM((320, tb), jnp.float32),
        ],
        compiler_params=pltpu.CompilerParams(
            dimension_semantics=("parallel",),
            vmem_limit_bytes=40 * 1024 * 1024,
        ),
    )(x_flat, w1b, b1c, w2b, b2c, fw1p, fb1c, fw2m, fb2c)


def _row_onehot(n_r):
    a = np.zeros((2, n_r, 5), np.float32)
    for d in range(2):
        for h in range(5):
            a[d, d + h, h] = 1.0
    return a


def _col_onehot(n_p, n_w):
    b = np.zeros((2 * n_p, n_w, 5), np.float32)
    for d in range(2):
        for p in range(n_p):
            for k in range(5):
                b[d * n_p + p, 2 * p + d + k, k] = 1.0
    return b


_A1 = _row_onehot(6)
_B1 = _col_onehot(12, 28)
_A2 = _row_onehot(6)
_B2 = _col_onehot(4, 12)


def _pack_conv1(w1):
    return jnp.einsum("drh,qwk,chk->dqcrw", _A1, _B1,
                      w1[:, 0]).reshape(480, 168)


def _pack_conv2(w2):
    return jnp.einsum("drh,qwk,oihk->dqorwi", _A2, _B2,
                      w2).reshape(320, 720)


@jax.jit
def _forward(x_nchw, w1, b1, w2, b2, fw1, fb1, fw2, fb2):
    n = x_nchw.shape[0]
    tb = _TB
    n_pad = ((n + tb - 1) // tb) * tb

    x = x_nchw.reshape(n, 784).astype(jnp.float32)
    if n_pad != n:
        x = jnp.pad(x, ((0, n_pad - n), (0, 0)))
    x_flat = x.T

    w1b = _pack_conv1(w1.astype(jnp.float32))
    b1c = jnp.tile(b1.astype(jnp.float32), 12).reshape(120, 1)
    w2b = _pack_conv2(w2.astype(jnp.float32))
    b2c = jnp.tile(b2.astype(jnp.float32), 16).reshape(320, 1)
    fw1p = (fw1.astype(jnp.float32).reshape(50, 20, 4, 4)
            .transpose(0, 2, 3, 1).reshape(50, 320))
    fb1c = fb1.reshape(50, 1).astype(jnp.float32)
    fw2m = fw2.astype(jnp.float32)
    fb2c = fb2.reshape(10, 1).astype(jnp.float32)

    out = _run(x_flat, w1b.astype(jnp.bfloat16), b1c,
               w2b.astype(jnp.bfloat16), b2c,
               fw1p.astype(jnp.bfloat16), fb1c,
               fw2m.astype(jnp.bfloat16), fb2c)
    return out[:, :n].T


def kernel(x_nchw, w1, b1, w2, b2, fw1, fb1, fw2, fb2):
    return _forward(x_nchw, w1, b1, w2, b2, fw1, fb1, fw2, fb2)


# speedup vs baseline: 1.7324x; 1.7023x over previous
import jax
import jax.numpy as jnp
import numpy as np
from jax.experimental import pallas as pl
from jax.experimental.pallas import tpu as pltpu

_TB = 1024


def _molu(x):
    return 0.5 * x * (1.0 + jnp.tanh(x))


def _fwd_kernel(x_ref, w1_ref, b1_ref, w2_ref, b2_ref,
                fw1_ref, fb1_ref, fw2_ref, fb2_ref,
                o_ref, xf_ref, y1_ref, f_ref):
    for r in range(28):
        xf_ref[28 * r:28 * (r + 1), :] = x_ref[r, :, :]

    for py in range(12):
        win = xf_ref[56 * py:56 * py + 168, :].astype(jnp.bfloat16)
        m = jnp.dot(w1_ref[...], win,
                    preferred_element_type=jnp.float32)
        p = jnp.maximum(m[0:240, :], m[240:480, :])
        p = jnp.maximum(p[0:120, :], p[120:240, :])
        y1_ref[120 * py:120 * (py + 1), :] = _molu(p + b1_ref[...]).astype(jnp.bfloat16)

    for py2 in range(4):
        win = y1_ref[240 * py2:240 * py2 + 720, :]
        m = jnp.dot(w2_ref[...], win,
                    preferred_element_type=jnp.float32)
        p = jnp.maximum(m[0:160, :], m[160:320, :])
        p = jnp.maximum(p[0:80, :], p[80:160, :])
        f_ref[80 * py2:80 * (py2 + 1), :] = p

    feats = _molu(f_ref[...] + b2_ref[...])

    h = _molu(jnp.dot(fw1_ref[...], feats.astype(jnp.bfloat16),
                      preferred_element_type=jnp.float32) + fb1_ref[...])
    logits = jnp.dot(fw2_ref[...], h.astype(jnp.bfloat16),
                     preferred_element_type=jnp.float32) + fb2_ref[...]
    mx = jnp.max(logits, axis=0, keepdims=True)
    sh = logits - mx
    lse = jnp.log(jnp.sum(jnp.exp(sh), axis=0, keepdims=True))
    o_ref[...] = (sh - lse).astype(o_ref.dtype)


def _run(x_flat, w1b, b1c, w2b, b2c, fw1p, fb1c, fw2m, fb2c):
    n_pad = x_flat.shape[-1]
    tb = _TB
    grid = (n_pad // tb,)
    return pl.pallas_call(
        _fwd_kernel,
        out_shape=jax.ShapeDtypeStruct((10, n_pad), jnp.float32),
        grid=grid,
        in_specs=[
            pl.BlockSpec((28, 28, tb), lambda i: (0, 0, i)),
            pl.BlockSpec((480, 168), lambda i: (0, 0)),
            pl.BlockSpec((120, 1), lambda i: (0, 0)),
            pl.BlockSpec((320, 720), lambda i: (0, 0)),
            pl.BlockSpec((320, 1), lambda i: (0, 0)),
            pl.BlockSpec((50, 320), lambda i: (0, 0)),
            pl.BlockSpec((50, 1), lambda i: (0, 0)),
            pl.BlockSpec((10, 50), lambda i: (0, 0)),
            pl.BlockSpec((10, 1), lambda i: (0, 0)),
        ],
        out_specs=pl.BlockSpec((10, tb), lambda i: (0, i)),
        scratch_shapes=[
            pltpu.VMEM((784, tb), jnp.float32),
            pltpu.VMEM((1440, tb), jnp.bfloat16),
            pltpu.VMEM((320, tb), jnp.float32),
        ],
        compiler_params=pltpu.CompilerParams(
            dimension_semantics=("parallel",),
            vmem_limit_bytes=40 * 1024 * 1024,
        ),
    )(x_flat, w1b, b1c, w2b, b2c, fw1p, fb1c, fw2m, fb2c)


def _row_onehot(n_r):
    a = np.zeros((2, n_r, 5), np.float32)
    for d in range(2):
        for h in range(5):
            a[d, d + h, h] = 1.0
    return a


def _col_onehot(n_p, n_w):
    b = np.zeros((2 * n_p, n_w, 5), np.float32)
    for d in range(2):
        for p in range(n_p):
            for k in range(5):
                b[d * n_p + p, 2 * p + d + k, k] = 1.0
    return b


_A1 = _row_onehot(6)
_B1 = _col_onehot(12, 28)
_A2 = _row_onehot(6)
_B2 = _col_onehot(4, 12)


def _pack_conv1(w1):
    return jnp.einsum("drh,qwk,chk->dqcrw", _A1, _B1,
                      w1[:, 0]).reshape(480, 168)


def _pack_conv2(w2):
    return jnp.einsum("drh,qwk,oihk->dqorwi", _A2, _B2,
                      w2).reshape(320, 720)


@jax.jit
def _forward(x_nchw, w1, b1, w2, b2, fw1, fb1, fw2, fb2):
    n = x_nchw.shape[0]
    tb = _TB
    n_pad = ((n + tb - 1) // tb) * tb

    x = x_nchw.reshape(n, 28, 28).astype(jnp.float32)
    if n_pad != n:
        x = jnp.pad(x, ((0, n_pad - n), (0, 0), (0, 0)))
    x_flat = x.transpose(1, 2, 0)

    w1b = _pack_conv1(w1.astype(jnp.float32))
    b1c = jnp.tile(b1.astype(jnp.float32), 12).reshape(120, 1)
    w2b = _pack_conv2(w2.astype(jnp.float32))
    b2c = jnp.tile(b2.astype(jnp.float32), 16).reshape(320, 1)
    fw1p = (fw1.astype(jnp.float32).reshape(50, 20, 4, 4)
            .transpose(0, 2, 3, 1).reshape(50, 320))
    fb1c = fb1.reshape(50, 1).astype(jnp.float32)
    fw2m = fw2.astype(jnp.float32)
    fb2c = fb2.reshape(10, 1).astype(jnp.float32)

    out = _run(x_flat, w1b.astype(jnp.bfloat16), b1c,
               w2b.astype(jnp.bfloat16), b2c,
               fw1p.astype(jnp.bfloat16), fb1c,
               fw2m.astype(jnp.bfloat16), fb2c)
    return out[:, :n].T


def kernel(x_nchw, w1, b1, w2, b2, fw1, fb1, fw2, fb2):
    return _forward(x_nchw, w1, b1, w2, b2, fw1, fb1, fw2, fb2)
